# baseline (speedup 1.0000x reference)
import functools

import jax
import jax.numpy as jnp
from jax import lax
from jax.experimental import pallas as pl
from jax.experimental.pallas import tpu as pltpu
from jax.experimental.pallas import tpu_sc as plsc

B = 16384
V = 1000000
D = 16
L = 16
NC, NS = 2, 16
NW = NC * NS
BPW = B // NW
CH = 128
NCH = BPW // CH
G = BPW // L

_mesh = plsc.VectorSubcoreMesh(core_axis_name="c", subcore_axis_name="s")


@functools.partial(
    pl.kernel,
    mesh=_mesh,
    out_type=jax.ShapeDtypeStruct((B,), jnp.float32),
    scratch_types=[
        pltpu.VMEM((BPW,), jnp.int32),
        pltpu.VMEM((BPW,), jnp.int32),
        pltpu.VMEM((D * BPW,), jnp.int32),
        pltpu.VMEM((D * BPW,), jnp.int32),
        pltpu.VMEM((D * BPW,), jnp.float32),
        pltpu.VMEM((D * BPW,), jnp.float32),
        pltpu.VMEM((BPW,), jnp.float32),
        pltpu.SemaphoreType.DMA,
    ],
    compiler_params=pltpu.CompilerParams(
        needs_layout_passes=False, use_tc_tiling_on_sc=False
    ),
)
def _sc_dot_kernel(uidx_hbm, iidx_hbm, utabf_hbm, itabf_hbm, out_hbm,
                   uidx_v, iidx_v, ueidx_v, ieidx_v, ubuf_v, ibuf_v,
                   scores_v, sem):
    wid = lax.axis_index("s") * NC + lax.axis_index("c")
    base = wid * BPW

    pltpu.sync_copy(uidx_hbm.at[pl.ds(base, BPW)], uidx_v)
    pltpu.sync_copy(iidx_hbm.at[pl.ds(base, BPW)], iidx_v)

    def build(g, carry):
        off = g * L
        uvec = uidx_v[pl.ds(off, L)]
        ivec = iidx_v[pl.ds(off, L)]
        for c in range(D):
            ueidx_v[pl.ds(c * BPW + off, L)] = uvec + c * V
            ieidx_v[pl.ds(c * BPW + off, L)] = ivec + c * V
        return carry

    lax.fori_loop(0, G, build, 0)

    copies = []
    for c in range(D):
        for ch in range(NCH):
            sl = pl.ds(c * BPW + ch * CH, CH)
            copies.append(
                pltpu.async_copy(utabf_hbm.at[ueidx_v.at[sl]], ubuf_v.at[sl], sem)
            )
            copies.append(
                pltpu.async_copy(itabf_hbm.at[ieidx_v.at[sl]], ibuf_v.at[sl], sem)
            )
    for cp in copies:
        cp.wait()

    def group(g, carry):
        off = g * L
        acc = jnp.zeros((L,), jnp.float32)
        for c in range(D):
            u = ubuf_v[pl.ds(c * BPW + off, L)]
            w = ibuf_v[pl.ds(c * BPW + off, L)]
            acc = acc + u * w
        scores_v[pl.ds(off, L)] = acc
        return carry

    lax.fori_loop(0, G, group, 0)

    pltpu.sync_copy(scores_v, out_hbm.at[pl.ds(base, BPW)])


def kernel(user_idx, item_idx, user_table, item_table):
    utabf = user_table.T.reshape(D * V)
    itabf = item_table.T.reshape(D * V)
    return _sc_dot_kernel(user_idx, item_idx, utabf, itabf)


# speedup vs baseline: 1.0024x; 1.0024x over previous
import functools

import jax
import jax.numpy as jnp
from jax import lax
from jax.experimental import pallas as pl
from jax.experimental.pallas import tpu as pltpu
from jax.experimental.pallas import tpu_sc as plsc

B = 16384
V = 1000000
D = 16
L = 16
NC, NS = 2, 16
NW = NC * NS
BPW = B // NW
CH = 128
NCH = BPW // CH
G = BPW // L

_mesh = plsc.VectorSubcoreMesh(core_axis_name="c", subcore_axis_name="s")


@functools.partial(
    pl.kernel,
    mesh=_mesh,
    out_type=jax.ShapeDtypeStruct((B,), jnp.float32),
    scratch_types=[
        pltpu.VMEM((BPW,), jnp.int32),
        pltpu.VMEM((BPW,), jnp.int32),
        pltpu.VMEM((D * BPW,), jnp.float32),
        pltpu.VMEM((D * BPW,), jnp.float32),
        pltpu.VMEM((BPW,), jnp.float32),
        pltpu.SemaphoreType.DMA,
    ],
    compiler_params=pltpu.CompilerParams(
        needs_layout_passes=False, use_tc_tiling_on_sc=False
    ),
)
def _sc_dot_kernel(uidx_hbm, iidx_hbm, utabf_hbm, itabf_hbm, out_hbm,
                   uidx_v, iidx_v, ubuf_v, ibuf_v, scores_v, sem):
    wid = lax.axis_index("s") * NC + lax.axis_index("c")
    base = wid * BPW

    pltpu.sync_copy(uidx_hbm.at[pl.ds(base, BPW)], uidx_v)
    pltpu.sync_copy(iidx_hbm.at[pl.ds(base, BPW)], iidx_v)

    copies = []
    for c in range(D):
        for ch in range(NCH):
            sl = pl.ds(c * BPW + ch * CH, CH)
            isl = pl.ds(ch * CH, CH)
            copies.append(
                pltpu.async_copy(
                    utabf_hbm.at[c].at[uidx_v.at[isl]], ubuf_v.at[sl], sem
                )
            )
            copies.append(
                pltpu.async_copy(
                    itabf_hbm.at[c].at[iidx_v.at[isl]], ibuf_v.at[sl], sem
                )
            )
    for cp in copies:
        cp.wait()

    def group(g, carry):
        off = g * L
        acc = jnp.zeros((L,), jnp.float32)
        for c in range(D):
            u = ubuf_v[pl.ds(c * BPW + off, L)]
            w = ibuf_v[pl.ds(c * BPW + off, L)]
            acc = acc + u * w
        scores_v[pl.ds(off, L)] = acc
        return carry

    lax.fori_loop(0, G, group, 0)

    pltpu.sync_copy(scores_v, out_hbm.at[pl.ds(base, BPW)])


def kernel(user_idx, item_idx, user_table, item_table):
    return _sc_dot_kernel(user_idx, item_idx, user_table.T, item_table.T)


# speedup vs baseline: 4.2097x; 4.1998x over previous
import functools

import jax
import jax.numpy as jnp
from jax import lax
from jax.experimental import pallas as pl
from jax.experimental.pallas import tpu as pltpu
from jax.experimental.pallas import tpu_sc as plsc

B = 16384
V = 1000000
D = 16
L = 16
NC, NS = 2, 16
NW = NC * NS
BPW = B // NW
CH = 128
NCH = BPW // CH
G = BPW // L

W = 2048
GRID = (V + W - 1) // W


def _detile_body(in_ref, *out_refs):
    x = in_ref[...]
    for c in range(D):
        out_refs[c][...] = x[c]


_detile = pl.pallas_call(
    _detile_body,
    grid=(GRID,),
    in_specs=[pl.BlockSpec((D, W), lambda i: (0, i))],
    out_specs=[pl.BlockSpec((W,), lambda i: (i,)) for _ in range(D)],
    out_shape=[jax.ShapeDtypeStruct((V,), jnp.float32) for _ in range(D)],
    compiler_params=pltpu.CompilerParams(
        dimension_semantics=("arbitrary",),
    ),
)


_mesh = plsc.VectorSubcoreMesh(core_axis_name="c", subcore_axis_name="s")


@functools.partial(
    pl.kernel,
    mesh=_mesh,
    out_type=jax.ShapeDtypeStruct((B,), jnp.float32),
    scratch_types=[
        pltpu.VMEM((BPW,), jnp.int32),
        pltpu.VMEM((BPW,), jnp.int32),
        pltpu.VMEM((D * BPW,), jnp.float32),
        pltpu.VMEM((D * BPW,), jnp.float32),
        pltpu.VMEM((BPW,), jnp.float32),
        pltpu.SemaphoreType.DMA,
    ],
    compiler_params=pltpu.CompilerParams(
        needs_layout_passes=False, use_tc_tiling_on_sc=False
    ),
)
def _sc_dot_kernel(uidx_hbm, iidx_hbm, *refs):
    ucol_hbm = refs[:D]
    icol_hbm = refs[D:2 * D]
    out_hbm = refs[2 * D]
    uidx_v, iidx_v, ubuf_v, ibuf_v, scores_v, sem = refs[2 * D + 1:]

    wid = lax.axis_index("s") * NC + lax.axis_index("c")
    base = wid * BPW

    pltpu.sync_copy(uidx_hbm.at[pl.ds(base, BPW)], uidx_v)
    pltpu.sync_copy(iidx_hbm.at[pl.ds(base, BPW)], iidx_v)

    copies = []
    for c in range(D):
        for ch in range(NCH):
            sl = pl.ds(c * BPW + ch * CH, CH)
            isl = pl.ds(ch * CH, CH)
            copies.append(
                pltpu.async_copy(ucol_hbm[c].at[uidx_v.at[isl]], ubuf_v.at[sl], sem)
            )
            copies.append(
                pltpu.async_copy(icol_hbm[c].at[iidx_v.at[isl]], ibuf_v.at[sl], sem)
            )
    for cp in copies:
        cp.wait()

    def group(g, carry):
        off = g * L
        acc = jnp.zeros((L,), jnp.float32)
        for c in range(D):
            u = ubuf_v[pl.ds(c * BPW + off, L)]
            w = ibuf_v[pl.ds(c * BPW + off, L)]
            acc = acc + u * w
        scores_v[pl.ds(off, L)] = acc
        return carry

    lax.fori_loop(0, G, group, 0)

    pltpu.sync_copy(scores_v, out_hbm.at[pl.ds(base, BPW)])


def kernel(user_idx, item_idx, user_table, item_table):
    ucols = _detile(user_table.T)
    icols = _detile(item_table.T)
    return _sc_dot_kernel(user_idx, item_idx, *ucols, *icols)
